# Initial kernel scaffold; baseline (speedup 1.0000x reference)
import jax
import jax.numpy as jnp
from jax.experimental import pallas as pl


def kernel(x, edge_index_0, edge_index_1, edge_weight_0, edge_weight_1, W_node, b_node, W_edge, b_edge):
    raise NotImplementedError("write your pallas kernel here")


# speedup vs baseline: 8.6855x; 8.6855x over previous
import functools

import jax
import jax.numpy as jnp
from jax import lax
from jax.experimental import pallas as pl
from jax.experimental.pallas import tpu as pltpu
from jax.experimental.pallas import tpu_sc as plsc

N = 512
F = 128
E = 16384
NN = N * N
NTILES = 16
SLICE = NN // NTILES
LANES = 16

_HIGH = jax.lax.Precision.HIGHEST


def _sc_build_adj_body(ei_hbm, w_hbm, a_hbm, asym_hbm, src_v, dst_v, w_v,
                       acc_a, acc_s):
    c = lax.axis_index("c")
    sid = lax.axis_index("s")
    lo = sid * SLICE

    pltpu.sync_copy(ei_hbm.at[c, 0], src_v)
    pltpu.sync_copy(ei_hbm.at[c, 1], dst_v)
    pltpu.sync_copy(w_hbm.at[c], w_v)

    zeros = jnp.zeros((LANES,), jnp.float32)

    def _zero(i, _):
        acc_a[pl.ds(i * LANES, LANES)] = zeros
        acc_s[pl.ds(i * LANES, LANES)] = zeros
        return 0

    lax.fori_loop(0, SLICE // LANES, _zero, 0)

    def _scatter(i, _):
        s16 = src_v[pl.ds(i * LANES, LANES)]
        d16 = dst_v[pl.ds(i * LANES, LANES)]
        w16 = w_v[pl.ds(i * LANES, LANES)]
        idx1 = s16 * N + d16
        idx2 = d16 * N + s16
        m1 = (idx1 >= lo) & (idx1 < lo + SLICE)
        m2 = (idx2 >= lo) & (idx2 < lo + SLICE)
        i1 = jnp.where(m1, idx1 - lo, 0)
        i2 = jnp.where(m2, idx2 - lo, 0)
        plsc.addupdate_scatter(acc_a, [i1], w16, mask=m1)
        plsc.addupdate_scatter(acc_s, [i1], w16, mask=m1)
        plsc.addupdate_scatter(acc_s, [i2], w16, mask=m2)
        return 0

    lax.fori_loop(0, E // LANES, _scatter, 0)

    pltpu.sync_copy(acc_a, a_hbm.at[c, pl.ds(lo, SLICE)])
    pltpu.sync_copy(acc_s, asym_hbm.at[c, pl.ds(lo, SLICE)])


def _sc_build_adj(ei_all, w_all):
    mesh = plsc.VectorSubcoreMesh(core_axis_name="c", subcore_axis_name="s")
    fn = functools.partial(
        pl.kernel,
        mesh=mesh,
        compiler_params=pltpu.CompilerParams(needs_layout_passes=False),
        out_type=[
            jax.ShapeDtypeStruct((2, NN), jnp.float32),
            jax.ShapeDtypeStruct((2, NN), jnp.float32),
        ],
        scratch_types=[
            pltpu.VMEM((E,), jnp.int32),
            pltpu.VMEM((E,), jnp.int32),
            pltpu.VMEM((E,), jnp.float32),
            pltpu.VMEM((SLICE,), jnp.float32),
            pltpu.VMEM((SLICE,), jnp.float32),
        ],
    )(_sc_build_adj_body)
    return fn(ei_all, w_all)


def _tc_dense_body(a_ref, s_ref, x_ref, wn_ref, bn_ref, we_ref, be_ref,
                   xn_ref, out_ref):
    x = x_ref[...]
    agg0 = lax.dot_general(a_ref[0], x, (((0,), (0,)), ((), ())),
                           precision=_HIGH, preferred_element_type=jnp.float32)
    agg1 = lax.dot_general(a_ref[1], x, (((0,), (0,)), ((), ())),
                           precision=_HIGH, preferred_element_type=jnp.float32)
    h0 = agg0 + x
    h1 = agg1 + x
    xn = (lax.dot_general(h0, wn_ref[:F, :], (((1,), (0,)), ((), ())),
                          precision=_HIGH, preferred_element_type=jnp.float32)
          + lax.dot_general(h1, wn_ref[F:, :], (((1,), (0,)), ((), ())),
                            precision=_HIGH, preferred_element_type=jnp.float32)
          + bn_ref[...][None, :])
    xn_ref[...] = xn

    ws = we_ref[2:2 + F, :] + we_ref[2 + F:, :]
    r = lax.dot_general(xn, ws, (((1,), (0,)), ((), ())),
                        precision=_HIGH, preferred_element_type=jnp.float32)
    rt = lax.dot_general(ws, xn, (((0,), (1,)), ((), ())),
                         precision=_HIGH, preferred_element_type=jnp.float32)
    for c in range(2):
        plane = (s_ref[0] * we_ref[0:1, c:c + 1]
                 + s_ref[1] * we_ref[1:2, c:c + 1]
                 + r[:, c:c + 1] + rt[c:c + 1, :]
                 + 2.0 * be_ref[c:c + 1])
        out_ref[c] = plane


def _tc_dense(a2, asym2, x, w_node, b_node, w_edge, b_edge):
    return pl.pallas_call(
        _tc_dense_body,
        out_shape=[
            jax.ShapeDtypeStruct((N, F), jnp.float32),
            jax.ShapeDtypeStruct((2, N, N), jnp.float32),
        ],
    )(a2, asym2, x, w_node, b_node, w_edge, b_edge)


def kernel(x, edge_index_0, edge_index_1, edge_weight_0, edge_weight_1,
           W_node, b_node, W_edge, b_edge):
    ei_all = jnp.stack([edge_index_0, edge_index_1])
    w_all = jnp.stack([edge_weight_0, edge_weight_1])
    a_flat, asym_flat = _sc_build_adj(ei_all, w_all)
    a2 = a_flat.reshape(2, N, N)
    asym2 = asym_flat.reshape(2, N, N)
    x_next, out2 = _tc_dense(a2, asym2, x, W_node, b_node, W_edge, b_edge)
    adj_out = jnp.transpose(out2, (1, 2, 0))
    return x_next, adj_out


# speedup vs baseline: 9.4604x; 1.0892x over previous
import functools

import jax
import jax.numpy as jnp
from jax import lax
from jax.experimental import pallas as pl
from jax.experimental.pallas import tpu as pltpu
from jax.experimental.pallas import tpu_sc as plsc

N = 512
F = 128
E = 16384
NTILES = 16
ROWS = N // NTILES
LANES = 16

_HIGH = jax.lax.Precision.HIGHEST


def _sc_build_adj_body(ei_hbm, w_hbm, a_hbm, src_v, dst_v, w_v, acc_a):
    c = lax.axis_index("c")
    sid = lax.axis_index("s")
    rlo = sid * ROWS

    pltpu.sync_copy(ei_hbm.at[pl.ds(c * 2 * E, E)], src_v)
    pltpu.sync_copy(ei_hbm.at[pl.ds(c * 2 * E + E, E)], dst_v)
    pltpu.sync_copy(w_hbm.at[pl.ds(c * E, E)], w_v)

    zeros = jnp.zeros((LANES,), jnp.float32)

    def _zero2(i, _):
        r = i // (N // LANES)
        k = i % (N // LANES)
        acc_a[r, pl.ds(k * LANES, LANES)] = zeros
        return 0

    lax.fori_loop(0, ROWS * (N // LANES), _zero2, 0)

    def _scatter(i, _):
        s16 = src_v[pl.ds(i * LANES, LANES)]
        d16 = dst_v[pl.ds(i * LANES, LANES)]
        w16 = w_v[pl.ds(i * LANES, LANES)]
        m = (s16 >= rlo) & (s16 < rlo + ROWS)
        r16 = jnp.where(m, s16 - rlo, 0)
        plsc.addupdate_scatter(acc_a, [r16, d16], w16, mask=m)
        return 0

    lax.fori_loop(0, E // LANES, _scatter, 0)

    pltpu.sync_copy(acc_a, a_hbm.at[c, pl.ds(rlo, ROWS), :])


def _sc_build_adj(ei_cat, w_cat):
    mesh = plsc.VectorSubcoreMesh(core_axis_name="c", subcore_axis_name="s")
    fn = functools.partial(
        pl.kernel,
        mesh=mesh,
        compiler_params=pltpu.CompilerParams(needs_layout_passes=False),
        out_type=jax.ShapeDtypeStruct((2, N, N), jnp.float32),
        scratch_types=[
            pltpu.VMEM((E,), jnp.int32),
            pltpu.VMEM((E,), jnp.int32),
            pltpu.VMEM((E,), jnp.float32),
            pltpu.VMEM((ROWS, N), jnp.float32),
        ],
    )(_sc_build_adj_body)
    return fn(ei_cat, w_cat)


def _tc_dense_body(a_ref, x_ref, wn_ref, bn_ref, we_ref, be_ref,
                   xn_ref, out_ref):
    x = x_ref[...]
    a0 = a_ref[0]
    a1 = a_ref[1]
    agg0 = lax.dot_general(a0, x, (((0,), (0,)), ((), ())),
                           precision=_HIGH, preferred_element_type=jnp.float32)
    agg1 = lax.dot_general(a1, x, (((0,), (0,)), ((), ())),
                           precision=_HIGH, preferred_element_type=jnp.float32)
    h0 = agg0 + x
    h1 = agg1 + x
    xn = (lax.dot_general(h0, wn_ref[:F, :], (((1,), (0,)), ((), ())),
                          precision=_HIGH, preferred_element_type=jnp.float32)
          + lax.dot_general(h1, wn_ref[F:, :], (((1,), (0,)), ((), ())),
                            precision=_HIGH, preferred_element_type=jnp.float32)
          + bn_ref[...][None, :])
    xn_ref[...] = xn

    s0 = a0 + jnp.swapaxes(a0, 0, 1)
    s1 = a1 + jnp.swapaxes(a1, 0, 1)
    ws = we_ref[2:2 + F, :] + we_ref[2 + F:, :]
    r = lax.dot_general(xn, ws, (((1,), (0,)), ((), ())),
                        precision=_HIGH, preferred_element_type=jnp.float32)
    rt = lax.dot_general(ws, xn, (((0,), (1,)), ((), ())),
                         precision=_HIGH, preferred_element_type=jnp.float32)
    for c in range(2):
        plane = (s0 * we_ref[0:1, c:c + 1]
                 + s1 * we_ref[1:2, c:c + 1]
                 + r[:, c:c + 1] + rt[c:c + 1, :]
                 + 2.0 * be_ref[c:c + 1])
        out_ref[c] = plane


def _tc_dense(a2, x, w_node, b_node, w_edge, b_edge):
    return pl.pallas_call(
        _tc_dense_body,
        out_shape=[
            jax.ShapeDtypeStruct((N, F), jnp.float32),
            jax.ShapeDtypeStruct((2, N, N), jnp.float32),
        ],
    )(a2, x, w_node, b_node, w_edge, b_edge)


def kernel(x, edge_index_0, edge_index_1, edge_weight_0, edge_weight_1,
           W_node, b_node, W_edge, b_edge):
    ei_cat = jnp.concatenate([edge_index_0.reshape(2 * E),
                              edge_index_1.reshape(2 * E)])
    w_cat = jnp.concatenate([edge_weight_0, edge_weight_1])
    a2 = _sc_build_adj(ei_cat, w_cat)
    x_next, out2 = _tc_dense(a2, x, W_node, b_node, W_edge, b_edge)
    adj_out = jnp.transpose(out2, (1, 2, 0))
    return x_next, adj_out


# speedup vs baseline: 12.7537x; 1.3481x over previous
import functools

import jax
import jax.numpy as jnp
from jax import lax
from jax.experimental import pallas as pl
from jax.experimental.pallas import tpu as pltpu
from jax.experimental.pallas import tpu_sc as plsc

N = 512
F = 128
E = 16384
NTILES = 16
ROWS = N // NTILES
LANES = 16

_HIGH = jax.lax.Precision.HIGHEST


EPW = E // NTILES
CHUNK = 128
NCHUNK = EPW // CHUNK
BAND = N * N // NTILES


def _sc_build_adj_body(ei_hbm, w_hbm, a_hbm, src_v, dst_v, val_v, idx_v,
                       zbuf, a_sh, sem):
    c = lax.axis_index("c")
    sid = lax.axis_index("s")
    ebase = c * 2 * E + sid * EPW

    pltpu.sync_copy(ei_hbm.at[pl.ds(ebase, EPW)], src_v)
    pltpu.sync_copy(ei_hbm.at[pl.ds(ebase + E, EPW)], dst_v)
    pltpu.sync_copy(w_hbm.at[c, sid], val_v)

    zeros = jnp.zeros((LANES,), jnp.float32)

    def _zero(i, _):
        for u in range(8):
            zbuf[pl.ds((i * 8 + u) * LANES, LANES)] = zeros
        return 0

    lax.fori_loop(0, BAND // LANES // 8, _zero, 0)
    flat_lo = sid * BAND
    pltpu.sync_copy(zbuf, a_sh.at[pl.ds(flat_lo, BAND)])
    plsc.subcore_barrier()

    for j in range(NCHUNK):
        for k in range(CHUNK // LANES):
            off = j * CHUNK + k * LANES
            s16 = src_v[pl.ds(off, LANES)]
            d16 = dst_v[pl.ds(off, LANES)]
            idx_v[j, pl.ds(k * LANES, LANES)] = s16 * N + d16

    ds = [
        pltpu.async_copy(val_v.at[j], a_sh.at[idx_v.at[j]], sem, add=True)
        for j in range(NCHUNK)
    ]
    for d in ds:
        d.wait()
    plsc.subcore_barrier()

    pltpu.sync_copy(a_sh.at[pl.ds(flat_lo, BAND)],
                    a_hbm.at[c, pl.ds(flat_lo, BAND)])


def _sc_build_adj(ei_cat, w_blk):
    mesh = plsc.VectorSubcoreMesh(core_axis_name="c", subcore_axis_name="s")
    fn = functools.partial(
        pl.kernel,
        mesh=mesh,
        compiler_params=pltpu.CompilerParams(needs_layout_passes=False),
        out_type=jax.ShapeDtypeStruct((2, N * N), jnp.float32),
        scratch_types=[
            pltpu.VMEM((EPW,), jnp.int32),
            pltpu.VMEM((EPW,), jnp.int32),
            pltpu.VMEM((NCHUNK, CHUNK), jnp.float32),
            pltpu.VMEM((NCHUNK, CHUNK), jnp.int32),
            pltpu.VMEM((BAND,), jnp.float32),
            pltpu.VMEM_SHARED((N * N,), jnp.float32),
            pltpu.SemaphoreType.DMA,
        ],
    )(_sc_build_adj_body)
    return fn(ei_cat, w_blk)


def _tc_dense_body(a_ref, x_ref, wn_ref, bn_ref, we_ref, be_ref,
                   xn_ref, out_ref):
    x = x_ref[...]
    a0 = a_ref[0]
    a1 = a_ref[1]
    agg0 = lax.dot_general(a0, x, (((0,), (0,)), ((), ())),
                           precision=_HIGH, preferred_element_type=jnp.float32)
    agg1 = lax.dot_general(a1, x, (((0,), (0,)), ((), ())),
                           precision=_HIGH, preferred_element_type=jnp.float32)
    h0 = agg0 + x
    h1 = agg1 + x
    xn = (lax.dot_general(h0, wn_ref[:F, :], (((1,), (0,)), ((), ())),
                          precision=_HIGH, preferred_element_type=jnp.float32)
          + lax.dot_general(h1, wn_ref[F:, :], (((1,), (0,)), ((), ())),
                            precision=_HIGH, preferred_element_type=jnp.float32)
          + bn_ref[...][None, :])
    xn_ref[...] = xn

    s0 = a0 + jnp.swapaxes(a0, 0, 1)
    s1 = a1 + jnp.swapaxes(a1, 0, 1)
    ws = we_ref[2:2 + F, :] + we_ref[2 + F:, :]
    r = lax.dot_general(xn, ws, (((1,), (0,)), ((), ())),
                        precision=_HIGH, preferred_element_type=jnp.float32)
    rt = lax.dot_general(ws, xn, (((0,), (1,)), ((), ())),
                         precision=_HIGH, preferred_element_type=jnp.float32)
    for c in range(2):
        plane = (s0 * we_ref[0:1, c:c + 1]
                 + s1 * we_ref[1:2, c:c + 1]
                 + r[:, c:c + 1] + rt[c:c + 1, :]
                 + 2.0 * be_ref[c:c + 1])
        out_ref[c] = plane


def _tc_dense(a2, x, w_node, b_node, w_edge, b_edge):
    return pl.pallas_call(
        _tc_dense_body,
        out_shape=[
            jax.ShapeDtypeStruct((N, F), jnp.float32),
            jax.ShapeDtypeStruct((2, N, N), jnp.float32),
        ],
    )(a2, x, w_node, b_node, w_edge, b_edge)


def kernel(x, edge_index_0, edge_index_1, edge_weight_0, edge_weight_1,
           W_node, b_node, W_edge, b_edge):
    ei_cat = jnp.concatenate([edge_index_0.reshape(2 * E),
                              edge_index_1.reshape(2 * E)])
    w_blk = jnp.stack([edge_weight_0, edge_weight_1]).reshape(
        2, NTILES, NCHUNK, CHUNK)
    a2 = _sc_build_adj(ei_cat, w_blk).reshape(2, N, N)
    x_next, out2 = _tc_dense(a2, x, W_node, b_node, W_edge, b_edge)
    adj_out = jnp.transpose(out2, (1, 2, 0))
    return x_next, adj_out


# speedup vs baseline: 13.8273x; 1.0842x over previous
import functools

import jax
import jax.numpy as jnp
from jax import lax
from jax.experimental import pallas as pl
from jax.experimental.pallas import tpu as pltpu
from jax.experimental.pallas import tpu_sc as plsc

N = 512
F = 128
E = 16384
NTILES = 16
ROWS = N // NTILES
LANES = 16

_HIGH = jax.lax.Precision.HIGHEST


EPW = E // NTILES
CHUNK = 128
NCHUNK = EPW // CHUNK
BAND = N * N // NTILES


def _sc_build_adj_body(ei_hbm, w_hbm, a_hbm, src_v, dst_v, val_v, idx_v,
                       zbuf, a_sh, sem):
    c = lax.axis_index("c")
    sid = lax.axis_index("s")
    ebase = c * 2 * E + sid * EPW

    pltpu.sync_copy(ei_hbm.at[pl.ds(ebase, EPW)], src_v)
    pltpu.sync_copy(ei_hbm.at[pl.ds(ebase + E, EPW)], dst_v)
    pltpu.sync_copy(w_hbm.at[c, sid], val_v)

    zeros = jnp.zeros((LANES,), jnp.float32)

    def _zero(i, _):
        for u in range(8):
            zbuf[pl.ds((i * 8 + u) * LANES, LANES)] = zeros
        return 0

    lax.fori_loop(0, BAND // LANES // 8, _zero, 0)
    flat_lo = sid * BAND
    pltpu.sync_copy(zbuf, a_sh.at[pl.ds(flat_lo, BAND)])

    for j in range(NCHUNK):
        for k in range(CHUNK // LANES):
            off = j * CHUNK + k * LANES
            s16 = src_v[pl.ds(off, LANES)]
            d16 = dst_v[pl.ds(off, LANES)]
            idx_v[j, pl.ds(k * LANES, LANES)] = s16 * N + d16
    plsc.subcore_barrier()

    ds = [
        pltpu.async_copy(val_v.at[j], a_sh.at[idx_v.at[j]], sem, add=True)
        for j in range(NCHUNK)
    ]
    for d in ds:
        d.wait()
    plsc.subcore_barrier()

    pltpu.sync_copy(a_sh.at[pl.ds(flat_lo, BAND)], zbuf)
    rlo = sid * ROWS
    wds = [
        pltpu.async_copy(zbuf.at[pl.ds(r * N, N)], a_hbm.at[c, rlo + r, :],
                         sem)
        for r in range(ROWS)
    ]
    for d in wds:
        d.wait()


def _sc_build_adj(ei_cat, w_blk):
    mesh = plsc.VectorSubcoreMesh(core_axis_name="c", subcore_axis_name="s")
    fn = functools.partial(
        pl.kernel,
        mesh=mesh,
        compiler_params=pltpu.CompilerParams(needs_layout_passes=False),
        out_type=jax.ShapeDtypeStruct((2, N, N), jnp.float32),
        scratch_types=[
            pltpu.VMEM((EPW,), jnp.int32),
            pltpu.VMEM((EPW,), jnp.int32),
            pltpu.VMEM((NCHUNK, CHUNK), jnp.float32),
            pltpu.VMEM((NCHUNK, CHUNK), jnp.int32),
            pltpu.VMEM((BAND,), jnp.float32),
            pltpu.VMEM_SHARED((N * N,), jnp.float32),
            pltpu.SemaphoreType.DMA,
        ],
    )(_sc_build_adj_body)
    return fn(ei_cat, w_blk)


def _tc_dense_body(a_ref, x_ref, wn_ref, bn_ref, we_ref, be_ref,
                   xn_ref, out_ref):
    x = x_ref[...]
    a0 = a_ref[0]
    a1 = a_ref[1]
    agg0 = lax.dot_general(a0, x, (((0,), (0,)), ((), ())),
                           precision=_HIGH, preferred_element_type=jnp.float32)
    agg1 = lax.dot_general(a1, x, (((0,), (0,)), ((), ())),
                           precision=_HIGH, preferred_element_type=jnp.float32)
    h0 = agg0 + x
    h1 = agg1 + x
    xn = (lax.dot_general(h0, wn_ref[:F, :], (((1,), (0,)), ((), ())),
                          precision=_HIGH, preferred_element_type=jnp.float32)
          + lax.dot_general(h1, wn_ref[F:, :], (((1,), (0,)), ((), ())),
                            precision=_HIGH, preferred_element_type=jnp.float32)
          + bn_ref[...][None, :])
    xn_ref[...] = xn

    s0 = a0 + jnp.swapaxes(a0, 0, 1)
    s1 = a1 + jnp.swapaxes(a1, 0, 1)
    ws = we_ref[2:2 + F, :] + we_ref[2 + F:, :]
    r = lax.dot_general(xn, ws, (((1,), (0,)), ((), ())),
                        precision=_HIGH, preferred_element_type=jnp.float32)
    rt = lax.dot_general(ws, xn, (((0,), (1,)), ((), ())),
                         precision=_HIGH, preferred_element_type=jnp.float32)
    for c in range(2):
        plane = (s0 * we_ref[0:1, c:c + 1]
                 + s1 * we_ref[1:2, c:c + 1]
                 + r[:, c:c + 1] + rt[c:c + 1, :]
                 + 2.0 * be_ref[c:c + 1])
        out_ref[c] = plane


def _tc_dense(a2, x, w_node, b_node, w_edge, b_edge):
    return pl.pallas_call(
        _tc_dense_body,
        out_shape=[
            jax.ShapeDtypeStruct((N, F), jnp.float32),
            jax.ShapeDtypeStruct((2, N, N), jnp.float32),
        ],
    )(a2, x, w_node, b_node, w_edge, b_edge)


def kernel(x, edge_index_0, edge_index_1, edge_weight_0, edge_weight_1,
           W_node, b_node, W_edge, b_edge):
    ei_cat = jnp.concatenate([edge_index_0.reshape(2 * E),
                              edge_index_1.reshape(2 * E)])
    w_blk = jnp.stack([edge_weight_0, edge_weight_1]).reshape(
        2, NTILES, NCHUNK, CHUNK)
    a2 = _sc_build_adj(ei_cat, w_blk)
    x_next, out2 = _tc_dense(a2, x, W_node, b_node, W_edge, b_edge)
    adj_out = jnp.transpose(out2, (1, 2, 0))
    return x_next, adj_out


# speedup vs baseline: 14.2784x; 1.0326x over previous
import functools

import jax
import jax.numpy as jnp
from jax import lax
from jax.experimental import pallas as pl
from jax.experimental.pallas import tpu as pltpu
from jax.experimental.pallas import tpu_sc as plsc

N = 512
F = 128
E = 16384
NTILES = 16
ROWS = N // NTILES
LANES = 16

_HIGH = jax.lax.Precision.HIGHEST


EPW = E // NTILES
CHUNK = 128
NCHUNK = EPW // CHUNK
BAND = N * N // NTILES


def _sc_build_adj_body(ei_hbm, w_hbm, a_hbm, src_v, dst_v, val_v, idx_v,
                       zbuf, a_sh, sem):
    c = lax.axis_index("c")
    sid = lax.axis_index("s")
    ebase = c * 2 * E + sid * EPW

    pltpu.sync_copy(ei_hbm.at[pl.ds(ebase, EPW)], src_v)
    pltpu.sync_copy(ei_hbm.at[pl.ds(ebase + E, EPW)], dst_v)
    pltpu.sync_copy(w_hbm.at[c, sid], val_v)

    zeros = jnp.zeros((LANES,), jnp.float32)

    def _zero(i, _):
        for u in range(8):
            zbuf[pl.ds((i * 8 + u) * LANES, LANES)] = zeros
        return 0

    lax.fori_loop(0, BAND // LANES // 8, _zero, 0)
    flat_lo = sid * BAND
    pltpu.sync_copy(zbuf, a_sh.at[pl.ds(flat_lo, BAND)])

    for j in range(NCHUNK):
        for k in range(CHUNK // LANES):
            off = j * CHUNK + k * LANES
            s16 = src_v[pl.ds(off, LANES)]
            d16 = dst_v[pl.ds(off, LANES)]
            idx_v[j, pl.ds(k * LANES, LANES)] = s16 * N + d16
    plsc.subcore_barrier()

    ds = [
        pltpu.async_copy(val_v.at[j], a_sh.at[idx_v.at[j]], sem, add=True)
        for j in range(NCHUNK)
    ]
    for d in ds:
        d.wait()
    plsc.subcore_barrier()

    pltpu.sync_copy(a_sh.at[pl.ds(flat_lo, BAND)], zbuf)
    rlo = sid * ROWS
    wds = [
        pltpu.async_copy(zbuf.at[pl.ds(r * N, N)], a_hbm.at[c, rlo + r, :],
                         sem)
        for r in range(ROWS)
    ]
    for d in wds:
        d.wait()


def _sc_build_adj(ei_cat, w_blk):
    mesh = plsc.VectorSubcoreMesh(core_axis_name="c", subcore_axis_name="s")
    fn = functools.partial(
        pl.kernel,
        mesh=mesh,
        compiler_params=pltpu.CompilerParams(needs_layout_passes=False),
        out_type=jax.ShapeDtypeStruct((2, N, N), jnp.float32),
        scratch_types=[
            pltpu.VMEM((EPW,), jnp.int32),
            pltpu.VMEM((EPW,), jnp.int32),
            pltpu.VMEM((NCHUNK, CHUNK), jnp.float32),
            pltpu.VMEM((NCHUNK, CHUNK), jnp.int32),
            pltpu.VMEM((BAND,), jnp.float32),
            pltpu.VMEM_SHARED((N * N,), jnp.float32),
            pltpu.SemaphoreType.DMA,
        ],
    )(_sc_build_adj_body)
    return fn(ei_cat, w_blk)


def _tc_dense_body(a_ref, x_ref, wn_ref, bn_ref, we_ref, be_ref,
                   xn_ref, out_ref):
    x = x_ref[...]
    a0 = a_ref[0]
    a1 = a_ref[1]
    agg0 = lax.dot_general(a0, x, (((0,), (0,)), ((), ())),
                           preferred_element_type=jnp.float32)
    agg1 = lax.dot_general(a1, x, (((0,), (0,)), ((), ())),
                           preferred_element_type=jnp.float32)
    h0 = agg0 + x
    h1 = agg1 + x
    xn = (lax.dot_general(h0, wn_ref[:F, :], (((1,), (0,)), ((), ())),
                          precision=_HIGH, preferred_element_type=jnp.float32)
          + lax.dot_general(h1, wn_ref[F:, :], (((1,), (0,)), ((), ())),
                            precision=_HIGH, preferred_element_type=jnp.float32)
          + bn_ref[...][None, :])
    xn_ref[...] = xn

    s0 = a0 + jnp.swapaxes(a0, 0, 1)
    s1 = a1 + jnp.swapaxes(a1, 0, 1)
    ws = we_ref[2:2 + F, :] + we_ref[2 + F:, :]
    r = lax.dot_general(xn, ws, (((1,), (0,)), ((), ())),
                        precision=_HIGH, preferred_element_type=jnp.float32)
    rt = lax.dot_general(ws, xn, (((0,), (1,)), ((), ())),
                         precision=_HIGH, preferred_element_type=jnp.float32)
    for c in range(2):
        plane = (s0 * we_ref[0:1, c:c + 1]
                 + s1 * we_ref[1:2, c:c + 1]
                 + r[:, c:c + 1] + rt[c:c + 1, :]
                 + 2.0 * be_ref[c:c + 1])
        out_ref[c] = plane


def _tc_dense(a2, x, w_node, b_node, w_edge, b_edge):
    return pl.pallas_call(
        _tc_dense_body,
        out_shape=[
            jax.ShapeDtypeStruct((N, F), jnp.float32),
            jax.ShapeDtypeStruct((2, N, N), jnp.float32),
        ],
    )(a2, x, w_node, b_node, w_edge, b_edge)


def kernel(x, edge_index_0, edge_index_1, edge_weight_0, edge_weight_1,
           W_node, b_node, W_edge, b_edge):
    ei_cat = jnp.concatenate([edge_index_0.reshape(2 * E),
                              edge_index_1.reshape(2 * E)])
    w_blk = jnp.concatenate([edge_weight_0, edge_weight_1]).reshape(
        2, NTILES, NCHUNK, CHUNK)
    a2 = _sc_build_adj(ei_cat, w_blk)
    x_next, out2 = _tc_dense(a2, x, W_node, b_node, W_edge, b_edge)
    adj_out = jnp.transpose(out2, (1, 2, 0))
    return x_next, adj_out
